# Initial kernel scaffold; baseline (speedup 1.0000x reference)
import functools

import jax
import jax.numpy as jnp
from jax import lax
from jax.experimental import pallas as pl
from jax.experimental.pallas import tpu as pltpu
from jax.experimental.pallas import tpu_sc as plsc

N = 10000
E = 320000
D = 128

NC = 2
NS = 16
NW = NC * NS

N_PAD = 10240
RPT = N_PAD // NS

CHUNK = 128
NCH = (E + NW * CHUNK - 1) // (NW * CHUNK)
EPT = NCH * CHUNK
E_PAD = EPT * NW

DEG_CH = EPT // 8

_mesh = plsc.VectorSubcoreMesh(core_axis_name="c", subcore_axis_name="s")


def _deg_body(dst_hbm, deg_out, hist, dbuf, hbuf, obuf, histall):
    c = lax.axis_index("c")
    s = lax.axis_index("s")
    wid = c * NS + s
    base = wid * EPT

    zeros16 = jnp.zeros((16,), jnp.float32)
    ones16 = jnp.ones((16,), jnp.float32)

    def zero_hist(i, _):
        hist[pl.ds(i * 16, 16)] = zeros16
        return _

    lax.fori_loop(0, N_PAD // 16, zero_hist, None)

    def outer(cb, _):
        pltpu.sync_copy(dst_hbm.at[pl.ds(base + cb * DEG_CH, DEG_CH)], dbuf)

        def inner(k, __):
            idx = dbuf[pl.ds(k * 16, 16)]
            plsc.addupdate_scatter(hist, [idx], ones16)
            return __

        lax.fori_loop(0, DEG_CH // 16, inner, None)
        return _

    lax.fori_loop(0, 8, outer, None)

    pltpu.sync_copy(hist, histall.at[s])
    plsc.subcore_barrier()
    pltpu.sync_copy(histall.at[:, pl.ds(s * RPT, RPT)], hbuf)

    def red(i, _):
        acc = hbuf[0, pl.ds(i * 16, 16)]
        for k in range(1, NS):
            acc = acc + hbuf[k, pl.ds(i * 16, 16)]
        obuf[pl.ds(i * 16, 16)] = acc
        return _

    lax.fori_loop(0, RPT // 16, red, None)
    pltpu.sync_copy(obuf, deg_out.at[c, pl.ds(s * RPT, RPT)])


_deg_call = pl.kernel(
    _deg_body,
    out_type=jax.ShapeDtypeStruct((NC, N_PAD), jnp.float32),
    mesh=_mesh,
    scratch_types=[
        pltpu.VMEM((N_PAD,), jnp.float32),
        pltpu.VMEM((DEG_CH,), jnp.int32),
        pltpu.VMEM((NS, RPT), jnp.float32),
        pltpu.VMEM((RPT,), jnp.float32),
        pltpu.VMEM_SHARED((NS, N_PAD), jnp.float32),
    ],
)


def _agg_body(hs_hbm, src_hbm, dst_hbm, part_out,
              sidx, didx, rows, zbuf, agg, gsem):
    c = lax.axis_index("c")
    s = lax.axis_index("s")
    wid = c * NS + s
    base = wid * EPT

    zeros16 = jnp.zeros((16,), jnp.float32)

    def zero_row(i, _):
        for k in range(D // 16):
            zbuf[i, pl.ds(k * 16, 16)] = zeros16
        return _

    lax.fori_loop(0, 128, zero_row, None)
    for k in range(RPT // 128):
        pltpu.sync_copy(zbuf, agg.at[pl.ds(s * RPT + k * 128, 128)])
    plsc.subcore_barrier()

    def step(j, _):
        be = base + j * CHUNK
        pltpu.sync_copy(src_hbm.at[pl.ds(be, CHUNK)], sidx)
        pltpu.async_copy(hs_hbm.at[sidx], rows, gsem).wait()
        pltpu.sync_copy(dst_hbm.at[pl.ds(be, CHUNK)], didx)
        pltpu.sync_copy(rows, agg.at[didx], add=True)
        return _

    lax.fori_loop(0, NCH, step, None)
    plsc.subcore_barrier()

    for k in range(RPT // 128):
        r0 = s * RPT + k * 128
        pltpu.sync_copy(agg.at[pl.ds(r0, 128)], part_out.at[c, pl.ds(r0, 128)])


_agg_call = pl.kernel(
    _agg_body,
    out_type=jax.ShapeDtypeStruct((NC, N_PAD, D), jnp.float32),
    mesh=_mesh,
    scratch_types=[
        pltpu.VMEM((CHUNK,), jnp.int32),
        pltpu.VMEM((CHUNK,), jnp.int32),
        pltpu.VMEM((CHUNK, D), jnp.float32),
        pltpu.VMEM((128, D), jnp.float32),
        pltpu.VMEM_SHARED((N_PAD, D), jnp.float32),
        pltpu.SemaphoreType.DMA,
    ],
)

_TC_R = 512


def _phase_a(x_ref, w_ref, degp_ref, hs_ref, dinv_ref):
    deg = degp_ref[0] + degp_ref[1] + 1.0
    dinv = lax.rsqrt(deg)
    h = jnp.dot(x_ref[...], w_ref[...], preferred_element_type=jnp.float32)
    hs_ref[...] = h * dinv[:, None]
    dinv_ref[...] = dinv


def _phase_b(agg_ref, hs_ref, dinv_ref, b_ref, w_ref, hs2_ref):
    a = agg_ref[0] + agg_ref[1]
    dinv = dinv_ref[...]
    h1 = jnp.maximum((a + hs_ref[...]) * dinv[:, None] + b_ref[...], 0.0)
    h2 = jnp.dot(h1, w_ref[...], preferred_element_type=jnp.float32)
    hs2_ref[...] = h2 * dinv[:, None]


def _phase_c(agg_ref, hs_ref, dinv_ref, b_ref, out_ref):
    a = agg_ref[0] + agg_ref[1]
    out_ref[...] = (a + hs_ref[...]) * dinv_ref[...][:, None] + b_ref[...]


_GRID = N_PAD // _TC_R

_phase_a_call = pl.pallas_call(
    _phase_a,
    grid=(_GRID,),
    in_specs=[
        pl.BlockSpec((_TC_R, D), lambda i: (i, 0)),
        pl.BlockSpec((D, D), lambda i: (0, 0)),
        pl.BlockSpec((NC, _TC_R), lambda i: (0, i)),
    ],
    out_specs=[
        pl.BlockSpec((_TC_R, D), lambda i: (i, 0)),
        pl.BlockSpec((_TC_R,), lambda i: (i,)),
    ],
    out_shape=[
        jax.ShapeDtypeStruct((N_PAD, D), jnp.float32),
        jax.ShapeDtypeStruct((N_PAD,), jnp.float32),
    ],
)

_phase_b_call = pl.pallas_call(
    _phase_b,
    grid=(_GRID,),
    in_specs=[
        pl.BlockSpec((NC, _TC_R, D), lambda i: (0, i, 0)),
        pl.BlockSpec((_TC_R, D), lambda i: (i, 0)),
        pl.BlockSpec((_TC_R,), lambda i: (i,)),
        pl.BlockSpec((D,), lambda i: (0,)),
        pl.BlockSpec((D, D), lambda i: (0, 0)),
    ],
    out_specs=pl.BlockSpec((_TC_R, D), lambda i: (i, 0)),
    out_shape=jax.ShapeDtypeStruct((N_PAD, D), jnp.float32),
)

_phase_c_call = pl.pallas_call(
    _phase_c,
    grid=(_GRID,),
    in_specs=[
        pl.BlockSpec((NC, _TC_R, D), lambda i: (0, i, 0)),
        pl.BlockSpec((_TC_R, D), lambda i: (i, 0)),
        pl.BlockSpec((_TC_R,), lambda i: (i,)),
        pl.BlockSpec((D,), lambda i: (0,)),
    ],
    out_specs=pl.BlockSpec((_TC_R, D), lambda i: (i, 0)),
    out_shape=jax.ShapeDtypeStruct((N_PAD, D), jnp.float32),
)


@jax.jit
def kernel(x, edge_index, W1, b1, W2, b2):
    src = edge_index[0]
    dst = edge_index[1]
    pad = E_PAD - E
    src_p = jnp.concatenate([src, jnp.zeros((pad,), jnp.int32)])
    dst_p = jnp.concatenate([dst, jnp.full((pad,), N, jnp.int32)])
    x_p = jnp.pad(x, ((0, N_PAD - N), (0, 0)))

    degp = _deg_call(dst_p)
    hs1, dinv = _phase_a_call(x_p, W1, degp)
    agg1 = _agg_call(hs1, src_p, dst_p)
    hs2 = _phase_b_call(agg1, hs1, dinv, b1, W2)
    agg2 = _agg_call(hs2, src_p, dst_p)
    out = _phase_c_call(agg2, hs2, dinv, b2)
    return out[:N]


# speedup vs baseline: 11.2068x; 11.2068x over previous
import functools

import jax
import jax.numpy as jnp
from jax import lax
from jax.experimental import pallas as pl
from jax.experimental.pallas import tpu as pltpu
from jax.experimental.pallas import tpu_sc as plsc

N = 10000
E = 320000
D = 128

NC = 2
NS = 16
NW = NC * NS

N_PAD = 10240
RPT = N_PAD // NS

CHUNK = 128
NCH = (E + NW * CHUNK - 1) // (NW * CHUNK)
EPT = NCH * CHUNK
E_PAD = EPT * NW

DEG_CH = EPT // 8

_mesh = plsc.VectorSubcoreMesh(core_axis_name="c", subcore_axis_name="s")


def _deg_body(dst_hbm, deg_out, hist, dbuf, hbuf, obuf, histall):
    c = lax.axis_index("c")
    s = lax.axis_index("s")
    wid = c * NS + s
    base = wid * EPT

    zeros16 = jnp.zeros((16,), jnp.float32)
    ones16 = jnp.ones((16,), jnp.float32)

    def zero_hist(i, _):
        hist[pl.ds(i * 16, 16)] = zeros16
        return _

    lax.fori_loop(0, N_PAD // 16, zero_hist, None)

    def outer(cb, _):
        pltpu.sync_copy(dst_hbm.at[pl.ds(base + cb * DEG_CH, DEG_CH)], dbuf)

        def inner(k, __):
            idx = dbuf[pl.ds(k * 16, 16)]
            plsc.addupdate_scatter(hist, [idx], ones16)
            return __

        lax.fori_loop(0, DEG_CH // 16, inner, None)
        return _

    lax.fori_loop(0, 8, outer, None)

    pltpu.sync_copy(hist, histall.at[s])
    plsc.subcore_barrier()
    pltpu.sync_copy(histall.at[:, pl.ds(s * RPT, RPT)], hbuf)

    def red(i, _):
        acc = hbuf[0, pl.ds(i * 16, 16)]
        for k in range(1, NS):
            acc = acc + hbuf[k, pl.ds(i * 16, 16)]
        obuf[pl.ds(i * 16, 16)] = acc
        return _

    lax.fori_loop(0, RPT // 16, red, None)
    pltpu.sync_copy(obuf, deg_out.at[c, pl.ds(s * RPT, RPT)])


_deg_call = pl.kernel(
    _deg_body,
    out_type=jax.ShapeDtypeStruct((NC, N_PAD), jnp.float32),
    mesh=_mesh,
    compiler_params=pltpu.CompilerParams(needs_layout_passes=False),
    scratch_types=[
        pltpu.VMEM((N_PAD,), jnp.float32),
        pltpu.VMEM((DEG_CH,), jnp.int32),
        pltpu.VMEM((NS, RPT), jnp.float32),
        pltpu.VMEM((RPT,), jnp.float32),
        pltpu.VMEM_SHARED((NS, N_PAD), jnp.float32),
    ],
)


def _agg_body(hs_hbm, src_hbm, dst_hbm, part_out,
              sidx, didx, rows, zbuf, agg, gsem):
    c = lax.axis_index("c")
    s = lax.axis_index("s")
    wid = c * NS + s
    base = wid * EPT

    zeros16 = jnp.zeros((16,), jnp.float32)

    def zero_row(i, _):
        for k in range(D // 16):
            zbuf[i, pl.ds(k * 16, 16)] = zeros16
        return _

    lax.fori_loop(0, 128, zero_row, None)
    for k in range(RPT // 128):
        pltpu.sync_copy(zbuf, agg.at[pl.ds(s * RPT + k * 128, 128)])
    plsc.subcore_barrier()

    def step(j, _):
        be = base + j * CHUNK
        pltpu.sync_copy(src_hbm.at[pl.ds(be, CHUNK)], sidx)
        pltpu.async_copy(hs_hbm.at[sidx], rows, gsem).wait()
        pltpu.sync_copy(dst_hbm.at[pl.ds(be, CHUNK)], didx)
        pltpu.sync_copy(rows, agg.at[didx], add=True)
        return _

    lax.fori_loop(0, NCH, step, None)
    plsc.subcore_barrier()

    for k in range(RPT // 128):
        r0 = s * RPT + k * 128
        pltpu.sync_copy(agg.at[pl.ds(r0, 128)], part_out.at[c, pl.ds(r0, 128)])


_agg_call = pl.kernel(
    _agg_body,
    out_type=jax.ShapeDtypeStruct((NC, N_PAD, D), jnp.float32),
    mesh=_mesh,
    scratch_types=[
        pltpu.VMEM((CHUNK,), jnp.int32),
        pltpu.VMEM((CHUNK,), jnp.int32),
        pltpu.VMEM((CHUNK, D), jnp.float32),
        pltpu.VMEM((128, D), jnp.float32),
        pltpu.VMEM_SHARED((N_PAD, D), jnp.float32),
        pltpu.SemaphoreType.DMA,
    ],
)

_TC_R = 512


def _phase_a(x_ref, w_ref, degp_ref, hs_ref, dinv_ref):
    deg = degp_ref[0] + degp_ref[1] + 1.0
    dinv = lax.rsqrt(deg)
    h = jnp.dot(x_ref[...], w_ref[...], preferred_element_type=jnp.float32)
    hs_ref[...] = h * dinv[:, None]
    dinv_ref[...] = dinv


def _phase_b(agg_ref, hs_ref, dinv_ref, b_ref, w_ref, hs2_ref):
    a = agg_ref[0] + agg_ref[1]
    dinv = dinv_ref[...]
    h1 = jnp.maximum((a + hs_ref[...]) * dinv[:, None] + b_ref[...], 0.0)
    h2 = jnp.dot(h1, w_ref[...], preferred_element_type=jnp.float32)
    hs2_ref[...] = h2 * dinv[:, None]


def _phase_c(agg_ref, hs_ref, dinv_ref, b_ref, out_ref):
    a = agg_ref[0] + agg_ref[1]
    out_ref[...] = (a + hs_ref[...]) * dinv_ref[...][:, None] + b_ref[...]


_GRID = N_PAD // _TC_R

_phase_a_call = pl.pallas_call(
    _phase_a,
    grid=(_GRID,),
    in_specs=[
        pl.BlockSpec((_TC_R, D), lambda i: (i, 0)),
        pl.BlockSpec((D, D), lambda i: (0, 0)),
        pl.BlockSpec((NC, _TC_R), lambda i: (0, i)),
    ],
    out_specs=[
        pl.BlockSpec((_TC_R, D), lambda i: (i, 0)),
        pl.BlockSpec((_TC_R,), lambda i: (i,)),
    ],
    out_shape=[
        jax.ShapeDtypeStruct((N_PAD, D), jnp.float32),
        jax.ShapeDtypeStruct((N_PAD,), jnp.float32),
    ],
)

_phase_b_call = pl.pallas_call(
    _phase_b,
    grid=(_GRID,),
    in_specs=[
        pl.BlockSpec((NC, _TC_R, D), lambda i: (0, i, 0)),
        pl.BlockSpec((_TC_R, D), lambda i: (i, 0)),
        pl.BlockSpec((_TC_R,), lambda i: (i,)),
        pl.BlockSpec((D,), lambda i: (0,)),
        pl.BlockSpec((D, D), lambda i: (0, 0)),
    ],
    out_specs=pl.BlockSpec((_TC_R, D), lambda i: (i, 0)),
    out_shape=jax.ShapeDtypeStruct((N_PAD, D), jnp.float32),
)

_phase_c_call = pl.pallas_call(
    _phase_c,
    grid=(_GRID,),
    in_specs=[
        pl.BlockSpec((NC, _TC_R, D), lambda i: (0, i, 0)),
        pl.BlockSpec((_TC_R, D), lambda i: (i, 0)),
        pl.BlockSpec((_TC_R,), lambda i: (i,)),
        pl.BlockSpec((D,), lambda i: (0,)),
    ],
    out_specs=pl.BlockSpec((_TC_R, D), lambda i: (i, 0)),
    out_shape=jax.ShapeDtypeStruct((N_PAD, D), jnp.float32),
)


@jax.jit
def kernel(x, edge_index, W1, b1, W2, b2):
    src = edge_index[0]
    dst = edge_index[1]
    pad = E_PAD - E
    src_p = jnp.concatenate([src, jnp.zeros((pad,), jnp.int32)])
    dst_p = jnp.concatenate([dst, jnp.full((pad,), N, jnp.int32)])
    x_p = jnp.pad(x, ((0, N_PAD - N), (0, 0)))

    degp = _deg_call(dst_p)
    hs1, dinv = _phase_a_call(x_p, W1, degp)
    agg1 = _agg_call(hs1, src_p, dst_p)
    hs2 = _phase_b_call(agg1, hs1, dinv, b1, W2)
    agg2 = _agg_call(hs2, src_p, dst_p)
    out = _phase_c_call(agg2, hs2, dinv, b2)
    return out[:N]
